# baseline (speedup 1.0000x reference)
import functools

import jax
import jax.numpy as jnp
from jax import lax
from jax.experimental import pallas as pl
from jax.experimental.pallas import tpu as pltpu
from jax.experimental.pallas import tpu_sc as plsc

_R = 4
_RP = 8
_D = 64
_SCALING = 1.0 / _R

_NC = 2
_NS = 16
_NW = _NC * _NS

_CHUNK = 128
_FIRE = 8
_LANES = 16
_CV = 8000
_TPR = 128 // _RP


def _transpose_kernel(v: int):
    n_chunks = v // _CV
    mesh = plsc.VectorSubcoreMesh(core_axis_name="c", subcore_axis_name="s")

    @functools.partial(
        pl.kernel,
        mesh=mesh,
        out_type=jax.ShapeDtypeStruct((v * _RP,), jnp.float32),
        scratch_types=[
            pltpu.VMEM((_R, _CV), jnp.float32),
            pltpu.VMEM((_CV * _RP,), jnp.float32),
            pltpu.SemaphoreType.DMA,
        ],
        compiler_params=pltpu.CompilerParams(
            use_tc_tiling_on_sc=False, needs_layout_passes=False
        ),
    )
    def transpose(a_hbm, table_hbm, buf_v, slab_v, sem):
        wid = lax.axis_index("s") * _NC + lax.axis_index("c")
        io8 = lax.broadcasted_iota(jnp.int32, (_LANES,), 0) * _RP

        zvec = jnp.zeros((_LANES,), jnp.float32)

        def zbody(k, carry):
            slab_v[pl.ds(k * _LANES, _LANES)] = zvec
            return carry

        lax.fori_loop(0, _CV * _RP // _LANES, zbody, 0, unroll=False)

        def do_chunk(c):
            for r in range(_R):
                pltpu.make_async_copy(
                    a_hbm.at[pl.ds(r * v + c * _CV, _CV)], buf_v.at[r], sem
                ).start()
            for r in range(_R):
                pltpu.make_async_copy(
                    a_hbm.at[pl.ds(0, _CV)], buf_v.at[0], sem
                ).wait()
            def qbody(q, carry):
                for r in range(_R):
                    plsc.store_scatter(
                        slab_v,
                        [io8 + (q * _LANES * _RP + r)],
                        buf_v[r, pl.ds(q * _LANES, _LANES)],
                    )
                return carry

            lax.fori_loop(0, _CV // _LANES, qbody, 0, unroll=False)
            pltpu.sync_copy(
                slab_v, table_hbm.at[pl.ds(c * _CV * _RP, _CV * _RP)]
            )

        for step in range(-(-n_chunks // _NW)):
            c = wid + step * _NW

            @pl.when(c < n_chunks)
            def _():
                do_chunk(c)

    return transpose


def _gather_kernel(n_tokens: int, v: int, l: int):
    b_per_w = n_tokens // _NW
    n_chunks = b_per_w // _CHUNK
    nb = b_per_w // l
    lp = l * _RP + (-(l * _RP) % 128)
    slab_n = nb * lp
    mesh = plsc.VectorSubcoreMesh(core_axis_name="c", subcore_axis_name="s")

    @functools.partial(
        pl.kernel,
        mesh=mesh,
        out_type=jax.ShapeDtypeStruct((_NW, slab_n), jnp.float32),
        scratch_types=[
            pltpu.VMEM((b_per_w,), jnp.int32),
            pltpu.VMEM((b_per_w, _RP), jnp.float32),
            pltpu.VMEM((slab_n,), jnp.float32),
            pltpu.SemaphoreType.DMA,
        ],
        compiler_params=pltpu.CompilerParams(
            use_tc_tiling_on_sc=False, needs_layout_passes=False
        ),
    )
    def gather(table_hbm, idx_hbm, out_hbm, idx_v, rows_v, slab_v, sem):
        wid = lax.axis_index("s") * _NC + lax.axis_index("c")
        base = wid * b_per_w
        pltpu.sync_copy(idx_hbm.at[pl.ds(base, b_per_w)], idx_v)

        zvec = jnp.zeros((_LANES,), jnp.float32)

        def zbody(k, carry):
            slab_v[pl.ds(k * _LANES, _LANES)] = zvec
            return carry

        lax.fori_loop(0, slab_n // _LANES, zbody, 0, unroll=False)

        def copy(j):
            return pltpu.make_async_copy(
                table_hbm.at[idx_v.at[pl.ds(j * _CHUNK, _CHUNK)]],
                rows_v.at[pl.ds(j * _CHUNK, _CHUNK)],
                sem,
            )

        def body(j, carry):
            copy(j).start()

            @pl.when(j >= _FIRE)
            def _():
                copy(j - _FIRE).wait()

            return carry

        lax.fori_loop(0, n_chunks, body, 0, unroll=False)
        for j in range(max(n_chunks - _FIRE, 0), n_chunks):
            copy(j).wait()

        io = lax.broadcasted_iota(jnp.int32, (_LANES,), 0)

        def rbody(g, carry):
            tvec = g * _LANES + io
            b_loc = tvec // l
            dbase = b_loc * lp + (tvec - b_loc * l) * _RP
            for r in range(_R):
                vals = plsc.load_gather(
                    rows_v, [tvec, jnp.full((_LANES,), r, jnp.int32)]
                )
                plsc.store_scatter(slab_v, [dbase + r], vals)
            return carry

        lax.fori_loop(0, b_per_w // _LANES, rbody, 0, unroll=False)
        pltpu.sync_copy(slab_v, out_hbm.at[wid])

    return gather


def _matmul_call(emb5, w, b: int, l: int, lp: int):
    grid = 8
    bb = b // grid
    n_cols = _TPR * _D
    nq = lp // 128

    def body(e_ref, w_ref, o_ref):
        for q in range(nq):
            tq = jnp.dot(
                e_ref[:, q * 128 : (q + 1) * 128],
                w_ref[...],
                preferred_element_type=jnp.float32,
            )
            for j in range(_TPR):
                li = q * _TPR + j
                if li < l:
                    o_ref[:, li, :] = tq[:, j * _D : (j + 1) * _D]

    return pl.pallas_call(
        body,
        grid=(grid,),
        in_specs=[
            pl.BlockSpec((bb, lp), lambda i: (i, 0)),
            pl.BlockSpec((128, n_cols), lambda i: (0, 0)),
        ],
        out_specs=pl.BlockSpec((bb, l, _D), lambda i: (i, 0, 0)),
        out_shape=jax.ShapeDtypeStruct((b, l, _D), jnp.float32),
    )(emb5, w)


def kernel(x, A, B):
    b, l = x.shape
    n_tokens = b * l
    v = A.shape[1]
    idx = x.reshape(n_tokens).astype(jnp.int32)
    a_flat = A.reshape(_R * v)
    bt8 = jnp.zeros((_RP, _D), jnp.float32).at[:_R, :].set(B.T * _SCALING)
    w = jnp.kron(jnp.eye(_TPR, dtype=jnp.float32), bt8)

    table = _transpose_kernel(v)(a_flat).reshape(v, _RP)
    lp = l * _RP + (-(l * _RP) % 128)
    emb = _gather_kernel(n_tokens, v, l)(table, idx)
    emb5 = emb.reshape(b, lp)
    return _matmul_call(emb5, w, b, l, lp)


# speedup vs baseline: 1.0679x; 1.0679x over previous
import functools

import jax
import jax.numpy as jnp
from jax import lax
from jax.experimental import pallas as pl
from jax.experimental.pallas import tpu as pltpu
from jax.experimental.pallas import tpu_sc as plsc

_R = 4
_RP = 8
_D = 64
_SCALING = 1.0 / _R

_NC = 2
_NS = 16
_NW = _NC * _NS

_CHUNK = 128
_FIRE = 8
_LANES = 16
_CV = 8000
_TPR = 128 // _RP


def _transpose_kernel(v: int):
    n_chunks = v // _CV
    mesh = plsc.VectorSubcoreMesh(core_axis_name="c", subcore_axis_name="s")

    @functools.partial(
        pl.kernel,
        mesh=mesh,
        out_type=jax.ShapeDtypeStruct((v * _RP,), jnp.float32),
        scratch_types=[
            pltpu.VMEM((_R, _CV), jnp.float32),
            pltpu.VMEM((_CV * _RP,), jnp.float32),
            pltpu.SemaphoreType.DMA,
        ],
        compiler_params=pltpu.CompilerParams(
            use_tc_tiling_on_sc=False, needs_layout_passes=False
        ),
    )
    def transpose(a_hbm, table_hbm, buf_v, slab_v, sem):
        wid = lax.axis_index("s") * _NC + lax.axis_index("c")
        io8 = lax.broadcasted_iota(jnp.int32, (_LANES,), 0) * _RP

        zvec = jnp.zeros((_LANES,), jnp.float32)

        def zbody(k, carry):
            slab_v[pl.ds(k * _LANES, _LANES)] = zvec
            return carry

        lax.fori_loop(0, _CV * _RP // _LANES, zbody, 0, unroll=False)

        def do_chunk(c):
            for r in range(_R):
                pltpu.make_async_copy(
                    a_hbm.at[pl.ds(r * v + c * _CV, _CV)], buf_v.at[r], sem
                ).start()
            for r in range(_R):
                pltpu.make_async_copy(
                    a_hbm.at[pl.ds(0, _CV)], buf_v.at[0], sem
                ).wait()
            def qbody(q, carry):
                for r in range(_R):
                    plsc.store_scatter(
                        slab_v,
                        [io8 + (q * _LANES * _RP + r)],
                        buf_v[r, pl.ds(q * _LANES, _LANES)],
                    )
                return carry

            lax.fori_loop(0, _CV // _LANES, qbody, 0, unroll=False)
            pltpu.sync_copy(
                slab_v, table_hbm.at[pl.ds(c * _CV * _RP, _CV * _RP)]
            )

        for step in range(-(-n_chunks // _NW)):
            c = wid + step * _NW

            @pl.when(c < n_chunks)
            def _():
                do_chunk(c)

    return transpose


def _lookup_kernel(n_tokens: int, v: int):
    b_per_w = n_tokens // _NW
    n_chunks = b_per_w // _CHUNK
    n_pairs = n_chunks // 2
    ob = _CHUNK * _D
    mesh = plsc.VectorSubcoreMesh(core_axis_name="c", subcore_axis_name="s")

    @functools.partial(
        pl.kernel,
        mesh=mesh,
        out_type=jax.ShapeDtypeStruct((n_tokens * _D,), jnp.float32),
        scratch_types=[
            pltpu.VMEM((b_per_w,), jnp.int32),
            pltpu.VMEM((2, _CHUNK, _RP), jnp.float32),
            pltpu.VMEM((2, ob), jnp.float32),
            pltpu.VMEM((_R * _D,), jnp.float32),
            pltpu.SemaphoreType.DMA,
            pltpu.SemaphoreType.DMA,
            pltpu.SemaphoreType.DMA,
            pltpu.SemaphoreType.DMA,
        ],
        compiler_params=pltpu.CompilerParams(
            use_tc_tiling_on_sc=False, needs_layout_passes=False
        ),
    )
    def lookup(
        table_hbm, idx_hbm, bt_hbm, out_hbm,
        idx_v, rows_v, oslab_v, bt_v, semg0, semg1, semo0, semo1,
    ):
        wid = lax.axis_index("s") * _NC + lax.axis_index("c")
        base = wid * b_per_w
        pltpu.sync_copy(idx_hbm.at[pl.ds(base, b_per_w)], idx_v)
        pltpu.sync_copy(bt_hbm, bt_v)
        bt = [
            [bt_v[pl.ds(r * _D + dq * _LANES, _LANES)] for dq in range(_D // _LANES)]
            for r in range(_R)
        ]
        cr = [jnp.full((_LANES,), r, jnp.int32) for r in range(_R)]

        def gather_copy(j, p, sem):
            return pltpu.make_async_copy(
                table_hbm.at[idx_v.at[pl.ds(j * _CHUNK, _CHUNK)]],
                rows_v.at[p],
                sem,
            )

        def out_copy(j, p, sem):
            return pltpu.make_async_copy(
                oslab_v.at[p],
                out_hbm.at[pl.ds(base * _D + j * ob, ob)],
                sem,
            )

        def compute(j, p):
            rows = rows_v.at[p]
            oslab = oslab_v.at[p]

            def tbody(t, carry):
                rs = jnp.full((_LANES,), t, jnp.int32)
                vals = [plsc.load_gather(rows, [rs, cr[r]]) for r in range(_R)]
                for dq in range(_D // _LANES):
                    o = vals[0] * bt[0][dq]
                    for r in range(1, _R):
                        o = o + vals[r] * bt[r][dq]
                    oslab[pl.ds(t * _D + dq * _LANES, _LANES)] = o
                return carry

            lax.fori_loop(0, _CHUNK, tbody, 0, unroll=2)

        def body(i, carry):
            a, b = 2 * i, 2 * i + 1
            gather_copy(a, 0, semg0).start()
            gather_copy(b, 1, semg1).start()

            @pl.when(i > 0)
            def _():
                out_copy(0, 0, semo0).wait()

            gather_copy(a, 0, semg0).wait()
            compute(a, 0)
            out_copy(a, 0, semo0).start()

            @pl.when(i > 0)
            def _():
                out_copy(0, 1, semo1).wait()

            gather_copy(b, 1, semg1).wait()
            compute(b, 1)
            out_copy(b, 1, semo1).start()
            return carry

        lax.fori_loop(0, n_pairs, body, 0, unroll=False)
        out_copy(0, 0, semo0).wait()
        out_copy(0, 1, semo1).wait()

    return lookup


def _gather_kernel(n_tokens: int, v: int, l: int):
    b_per_w = n_tokens // _NW
    n_chunks = b_per_w // _CHUNK
    nb = b_per_w // l
    lp = l * _RP + (-(l * _RP) % 128)
    slab_n = nb * lp
    mesh = plsc.VectorSubcoreMesh(core_axis_name="c", subcore_axis_name="s")

    @functools.partial(
        pl.kernel,
        mesh=mesh,
        out_type=jax.ShapeDtypeStruct((_NW, slab_n), jnp.float32),
        scratch_types=[
            pltpu.VMEM((b_per_w,), jnp.int32),
            pltpu.VMEM((b_per_w, _RP), jnp.float32),
            pltpu.VMEM((slab_n,), jnp.float32),
            pltpu.SemaphoreType.DMA,
        ],
        compiler_params=pltpu.CompilerParams(
            use_tc_tiling_on_sc=False, needs_layout_passes=False
        ),
    )
    def gather(table_hbm, idx_hbm, out_hbm, idx_v, rows_v, slab_v, sem):
        wid = lax.axis_index("s") * _NC + lax.axis_index("c")
        base = wid * b_per_w
        pltpu.sync_copy(idx_hbm.at[pl.ds(base, b_per_w)], idx_v)

        zvec = jnp.zeros((_LANES,), jnp.float32)

        def zbody(k, carry):
            slab_v[pl.ds(k * _LANES, _LANES)] = zvec
            return carry

        lax.fori_loop(0, slab_n // _LANES, zbody, 0, unroll=False)

        def copy(j):
            return pltpu.make_async_copy(
                table_hbm.at[idx_v.at[pl.ds(j * _CHUNK, _CHUNK)]],
                rows_v.at[pl.ds(j * _CHUNK, _CHUNK)],
                sem,
            )

        def body(j, carry):
            copy(j).start()

            @pl.when(j >= _FIRE)
            def _():
                copy(j - _FIRE).wait()

            return carry

        lax.fori_loop(0, n_chunks, body, 0, unroll=False)
        for j in range(max(n_chunks - _FIRE, 0), n_chunks):
            copy(j).wait()

        io = lax.broadcasted_iota(jnp.int32, (_LANES,), 0)

        def rbody(g, carry):
            tvec = g * _LANES + io
            b_loc = tvec // l
            dbase = b_loc * lp + (tvec - b_loc * l) * _RP
            for r in range(_R):
                vals = plsc.load_gather(
                    rows_v, [tvec, jnp.full((_LANES,), r, jnp.int32)]
                )
                plsc.store_scatter(slab_v, [dbase + r], vals)
            return carry

        lax.fori_loop(0, b_per_w // _LANES, rbody, 0, unroll=False)
        pltpu.sync_copy(slab_v, out_hbm.at[wid])

    return gather


def _matmul_call(emb5, w, b: int, l: int, lp: int):
    grid = 8
    bb = b // grid
    n_cols = _TPR * _D
    nq = lp // 128

    def body(e_ref, w_ref, o_ref):
        for q in range(nq):
            tq = jnp.dot(
                e_ref[:, q * 128 : (q + 1) * 128],
                w_ref[...],
                preferred_element_type=jnp.float32,
            )
            for j in range(_TPR):
                li = q * _TPR + j
                if li < l:
                    o_ref[:, li, :] = tq[:, j * _D : (j + 1) * _D]

    return pl.pallas_call(
        body,
        grid=(grid,),
        in_specs=[
            pl.BlockSpec((bb, lp), lambda i: (i, 0)),
            pl.BlockSpec((128, n_cols), lambda i: (0, 0)),
        ],
        out_specs=pl.BlockSpec((bb, l, _D), lambda i: (i, 0, 0)),
        out_shape=jax.ShapeDtypeStruct((b, l, _D), jnp.float32),
    )(emb5, w)


def kernel(x, A, B):
    b, l = x.shape
    n_tokens = b * l
    v = A.shape[1]
    idx = x.reshape(n_tokens).astype(jnp.int32)
    a_flat = A.reshape(_R * v)
    bt8 = jnp.zeros((_RP, _D), jnp.float32).at[:_R, :].set(B.T * _SCALING)
    w = jnp.kron(jnp.eye(_TPR, dtype=jnp.float32), bt8)

    btf = (B.T * _SCALING).reshape(_R * _D)
    table = _transpose_kernel(v)(a_flat).reshape(v, _RP)
    out_flat = _lookup_kernel(n_tokens, v)(table, idx, btf)
    return out_flat.reshape(b, l, _D)


# speedup vs baseline: 1.2406x; 1.1617x over previous
import functools

import jax
import jax.numpy as jnp
from jax import lax
from jax.experimental import pallas as pl
from jax.experimental.pallas import tpu as pltpu
from jax.experimental.pallas import tpu_sc as plsc

_R = 4
_RP = 8
_D = 64
_SCALING = 1.0 / _R

_NC = 2
_NS = 16
_NW = _NC * _NS

_CHUNK = 128
_FIRE = 8
_LANES = 16
_CV = 8000
_TPR = 128 // _RP


def _transpose_kernel(v: int):
    n_chunks = v // _CV
    mesh = plsc.VectorSubcoreMesh(core_axis_name="c", subcore_axis_name="s")

    @functools.partial(
        pl.kernel,
        mesh=mesh,
        out_type=jax.ShapeDtypeStruct((v * _RP,), jnp.float32),
        scratch_types=[
            pltpu.VMEM((_R, _CV), jnp.float32),
            pltpu.VMEM((_CV * _RP,), jnp.float32),
            pltpu.SemaphoreType.DMA,
        ],
        compiler_params=pltpu.CompilerParams(
            use_tc_tiling_on_sc=False, needs_layout_passes=False
        ),
    )
    def transpose(a_hbm, table_hbm, buf_v, slab_v, sem):
        wid = lax.axis_index("s") * _NC + lax.axis_index("c")
        io8 = lax.broadcasted_iota(jnp.int32, (_LANES,), 0) * _RP

        zvec = jnp.zeros((_LANES,), jnp.float32)

        def zbody(k, carry):
            slab_v[pl.ds(k * _LANES, _LANES)] = zvec
            return carry

        lax.fori_loop(0, _CV * _RP // _LANES, zbody, 0, unroll=False)

        def do_chunk(c):
            for r in range(_R):
                pltpu.make_async_copy(
                    a_hbm.at[pl.ds(r * v + c * _CV, _CV)], buf_v.at[r], sem
                ).start()
            for r in range(_R):
                pltpu.make_async_copy(
                    a_hbm.at[pl.ds(0, _CV)], buf_v.at[0], sem
                ).wait()

            def qbody(q, carry):
                for r in range(_R):
                    plsc.store_scatter(
                        slab_v,
                        [io8 + (q * _LANES * _RP + r)],
                        buf_v[r, pl.ds(q * _LANES, _LANES)],
                    )
                return carry

            lax.fori_loop(0, _CV // _LANES, qbody, 0, unroll=False)
            pltpu.sync_copy(
                slab_v, table_hbm.at[pl.ds(c * _CV * _RP, _CV * _RP)]
            )

        for step in range(-(-n_chunks // _NW)):
            c = wid + step * _NW

            @pl.when(c < n_chunks)
            def _():
                do_chunk(c)

    return transpose


def _gather_kernel(n_tokens: int, v: int):
    b_per_w = n_tokens // _NW
    n_chunks = b_per_w // _CHUNK
    mesh = plsc.VectorSubcoreMesh(core_axis_name="c", subcore_axis_name="s")

    @functools.partial(
        pl.kernel,
        mesh=mesh,
        out_type=jax.ShapeDtypeStruct((_NW, b_per_w, _RP), jnp.float32),
        scratch_types=[
            pltpu.VMEM((b_per_w,), jnp.int32),
            pltpu.VMEM((b_per_w, _RP), jnp.float32),
            pltpu.SemaphoreType.DMA,
        ],
        compiler_params=pltpu.CompilerParams(
            use_tc_tiling_on_sc=False, needs_layout_passes=False
        ),
    )
    def gather(table_hbm, idx_hbm, out_hbm, idx_v, rows_v, sem):
        wid = lax.axis_index("s") * _NC + lax.axis_index("c")
        base = wid * b_per_w
        pltpu.sync_copy(idx_hbm.at[pl.ds(base, b_per_w)], idx_v)

        def copy(j):
            return pltpu.make_async_copy(
                table_hbm.at[idx_v.at[pl.ds(j * _CHUNK, _CHUNK)]],
                rows_v.at[pl.ds(j * _CHUNK, _CHUNK)],
                sem,
            )

        def body(j, carry):
            copy(j).start()

            @pl.when(j >= _FIRE)
            def _():
                copy(j - _FIRE).wait()

            return carry

        lax.fori_loop(0, n_chunks, body, 0, unroll=False)
        for j in range(max(n_chunks - _FIRE, 0), n_chunks):
            copy(j).wait()
        pltpu.sync_copy(rows_v, out_hbm.at[wid])

    return gather


def _matmul_call(emb2, w, n_rows: int):
    grid = 8
    bm = n_rows // grid
    n_cols = _TPR * _D

    def body(e_ref, w_ref, o_ref):
        e = e_ref[...]
        for q in range(n_cols // 128):
            o_ref[:, q, :] = jnp.dot(
                e,
                w_ref[:, q * 128 : (q + 1) * 128],
                preferred_element_type=jnp.float32,
            )

    return pl.pallas_call(
        body,
        grid=(grid,),
        in_specs=[
            pl.BlockSpec((bm, 128), lambda i: (i, 0)),
            pl.BlockSpec((128, n_cols), lambda i: (0, 0)),
        ],
        out_specs=pl.BlockSpec((bm, n_cols // 128, 128), lambda i: (i, 0, 0)),
        out_shape=jax.ShapeDtypeStruct((n_rows, n_cols // 128, 128), jnp.float32),
    )(emb2, w)


def kernel(x, A, B):
    b, l = x.shape
    n_tokens = b * l
    v = A.shape[1]
    idx = x.reshape(n_tokens).astype(jnp.int32)
    a_flat = A.reshape(_R * v)
    bt8 = jnp.zeros((_RP, _D), jnp.float32).at[:_R, :].set(B.T * _SCALING)
    w = jnp.kron(jnp.eye(_TPR, dtype=jnp.float32), bt8)

    table = _transpose_kernel(v)(a_flat).reshape(v, _RP)
    emb = _gather_kernel(n_tokens, v)(table, idx)
    n_rows = n_tokens * _RP // 128
    emb2 = emb.reshape(n_rows, 128)
    out = _matmul_call(emb2, w, n_rows)
    return out.reshape(b, l, _D)


# speedup vs baseline: 1.3052x; 1.0520x over previous
import functools

import jax
import jax.numpy as jnp
from jax import lax
from jax.experimental import pallas as pl
from jax.experimental.pallas import tpu as pltpu
from jax.experimental.pallas import tpu_sc as plsc

_R = 4
_RP = 8
_D = 64
_SCALING = 1.0 / _R

_NC = 2
_NS = 16
_NW = _NC * _NS

_CHUNK = 128
_FIRE = 8
_LANES = 16
_CV = 8000
_TPR = 128 // _RP


def _transpose_kernel(v: int):
    n_chunks = v // _CV
    mesh = plsc.VectorSubcoreMesh(core_axis_name="c", subcore_axis_name="s")

    @functools.partial(
        pl.kernel,
        mesh=mesh,
        out_type=jax.ShapeDtypeStruct((v * _RP,), jnp.float32),
        scratch_types=[
            pltpu.VMEM((_R, _CV), jnp.float32),
            pltpu.VMEM((_CV * _RP,), jnp.float32),
            pltpu.SemaphoreType.DMA,
        ],
        compiler_params=pltpu.CompilerParams(
            use_tc_tiling_on_sc=False, needs_layout_passes=False
        ),
    )
    def transpose(a_hbm, table_hbm, buf_v, slab_v, sem):
        wid = lax.axis_index("s") * _NC + lax.axis_index("c")
        io8 = lax.broadcasted_iota(jnp.int32, (_LANES,), 0) * _RP

        zvec = jnp.zeros((_LANES,), jnp.float32)

        def zbody(k, carry):
            slab_v[pl.ds(k * _LANES, _LANES)] = zvec
            return carry

        lax.fori_loop(0, _CV * _RP // _LANES, zbody, 0, unroll=False)

        def do_chunk(c):
            for r in range(_R):
                pltpu.make_async_copy(
                    a_hbm.at[r, pl.ds(c * _CV, _CV)], buf_v.at[r], sem
                ).start()
            for r in range(_R):
                pltpu.make_async_copy(
                    a_hbm.at[0, pl.ds(0, _CV)], buf_v.at[0], sem
                ).wait()

            def qbody(q, carry):
                for r in range(_R):
                    plsc.store_scatter(
                        slab_v,
                        [io8 + (q * _LANES * _RP + r)],
                        buf_v[r, pl.ds(q * _LANES, _LANES)],
                    )
                return carry

            lax.fori_loop(0, _CV // _LANES, qbody, 0, unroll=False)
            pltpu.sync_copy(
                slab_v, table_hbm.at[pl.ds(c * _CV * _RP, _CV * _RP)]
            )

        for step in range(-(-n_chunks // _NW)):
            c = wid + step * _NW

            @pl.when(c < n_chunks)
            def _():
                do_chunk(c)

    return transpose


def _gather_kernel(n_tokens: int, v: int):
    b_per_w = n_tokens // _NW
    n_chunks = b_per_w // _CHUNK
    mesh = plsc.VectorSubcoreMesh(core_axis_name="c", subcore_axis_name="s")

    @functools.partial(
        pl.kernel,
        mesh=mesh,
        out_type=jax.ShapeDtypeStruct((_NW, b_per_w, _RP), jnp.float32),
        scratch_types=[
            pltpu.VMEM((b_per_w,), jnp.int32),
            pltpu.VMEM((b_per_w, _RP), jnp.float32),
            pltpu.SemaphoreType.DMA,
        ],
        compiler_params=pltpu.CompilerParams(
            use_tc_tiling_on_sc=False, needs_layout_passes=False
        ),
    )
    def gather(table_hbm, idx_hbm, out_hbm, idx_v, rows_v, sem):
        wid = lax.axis_index("s") * _NC + lax.axis_index("c")
        base = wid * b_per_w
        pltpu.sync_copy(idx_hbm.at[pl.ds(base, b_per_w)], idx_v)

        def copy(j):
            return pltpu.make_async_copy(
                table_hbm.at[idx_v.at[pl.ds(j * _CHUNK, _CHUNK)]],
                rows_v.at[pl.ds(j * _CHUNK, _CHUNK)],
                sem,
            )

        def body(j, carry):
            copy(j).start()

            @pl.when(j >= _FIRE)
            def _():
                copy(j - _FIRE).wait()

            return carry

        lax.fori_loop(0, n_chunks, body, 0, unroll=False)
        for j in range(max(n_chunks - _FIRE, 0), n_chunks):
            copy(j).wait()
        pltpu.sync_copy(rows_v, out_hbm.at[wid])

    return gather


def _matmul_call(emb2, w, n_rows: int):
    grid = 8
    bm = n_rows // grid
    n_cols = _TPR * _D

    def body(e_ref, w_ref, o_ref):
        o_ref[...] = jnp.dot(
            e_ref[...], w_ref[...], preferred_element_type=jnp.float32
        )

    return pl.pallas_call(
        body,
        grid=(grid,),
        in_specs=[
            pl.BlockSpec((bm, 128), lambda i: (i, 0)),
            pl.BlockSpec((128, n_cols), lambda i: (0, 0)),
        ],
        out_specs=pl.BlockSpec((bm, n_cols), lambda i: (i, 0)),
        out_shape=jax.ShapeDtypeStruct((n_rows, n_cols), jnp.float32),
    )(emb2, w)


def kernel(x, A, B):
    b, l = x.shape
    n_tokens = b * l
    v = A.shape[1]
    idx = x.reshape(n_tokens).astype(jnp.int32)
    bt8 = jnp.zeros((_RP, _D), jnp.float32).at[:_R, :].set(B.T * _SCALING)
    w = jnp.kron(jnp.eye(_TPR, dtype=jnp.float32), bt8)

    table = _transpose_kernel(v)(A).reshape(v, _RP)
    emb = _gather_kernel(n_tokens, v)(table, idx)
    n_rows = n_tokens * _RP // 128
    emb2 = emb.reshape(n_rows, 128)
    out = _matmul_call(emb2, w, n_rows)
    return out.reshape(b, l, _D)


# speedup vs baseline: 1.4222x; 1.0896x over previous
import functools

import jax
import jax.numpy as jnp
from jax import lax
from jax.experimental import pallas as pl
from jax.experimental.pallas import tpu as pltpu
from jax.experimental.pallas import tpu_sc as plsc

_R = 4
_RP = 8
_D = 64
_SCALING = 1.0 / _R

_NC = 2
_NS = 16
_NW = _NC * _NS

_CHUNK = 128
_FIRE = 8
_LANES = 16
_CV = 4000
_TPR = 128 // _RP


def _transpose_kernel(v: int):
    n_chunks = v // _CV
    n_steps = -(-n_chunks // _NW)
    mesh = plsc.VectorSubcoreMesh(core_axis_name="c", subcore_axis_name="s")

    @functools.partial(
        pl.kernel,
        mesh=mesh,
        out_type=jax.ShapeDtypeStruct((v * _RP,), jnp.float32),
        scratch_types=[
            pltpu.VMEM((2, _R, _CV), jnp.float32),
            pltpu.VMEM((2, _CV * _RP), jnp.float32),
            pltpu.SemaphoreType.DMA,
            pltpu.SemaphoreType.DMA,
            pltpu.SemaphoreType.DMA,
            pltpu.SemaphoreType.DMA,
        ],
        compiler_params=pltpu.CompilerParams(
            use_tc_tiling_on_sc=False, needs_layout_passes=False
        ),
    )
    def transpose(a_hbm, table_hbm, buf_v, slab_v, semg0, semg1, semo0, semo1):
        wid = lax.axis_index("s") * _NC + lax.axis_index("c")
        io8 = lax.broadcasted_iota(jnp.int32, (_LANES,), 0) * _RP
        semg = [semg0, semg1]
        semo = [semo0, semo1]

        zvec = jnp.zeros((_LANES,), jnp.float32)

        def zbody(k, carry):
            for p in range(2):
                slab_v[p, pl.ds(k * _LANES, _LANES)] = zvec
            return carry

        lax.fori_loop(0, _CV * _RP // _LANES, zbody, 0, unroll=False)

        def chunk_of(s):
            return wid + s * _NW

        def load_copy(c, p, r):
            return pltpu.make_async_copy(
                a_hbm.at[r, pl.ds(c * _CV, _CV)], buf_v.at[p, r], semg[p]
            )

        def out_copy(c, p):
            return pltpu.make_async_copy(
                slab_v.at[p],
                table_hbm.at[pl.ds(c * _CV * _RP, _CV * _RP)],
                semo[p],
            )

        def fire_loads(s, p):
            c = chunk_of(s)

            @pl.when(c < n_chunks)
            def _():
                for r in range(_R):
                    load_copy(c, p, r).start()

        fire_loads(0, 0)
        for s in range(n_steps):
            p = s % 2
            c = chunk_of(s)
            fire_loads(s + 1, 1 - p)

            @pl.when(c < n_chunks)
            def _():
                for r in range(_R):
                    load_copy(c, p, r).wait()
                if s >= 2:
                    out_copy(c, p).wait()

                def qbody(q, carry):
                    for r in range(_R):
                        plsc.store_scatter(
                            slab_v.at[p],
                            [io8 + (q * _LANES * _RP + r)],
                            buf_v[p, r, pl.ds(q * _LANES, _LANES)],
                        )
                    return carry

                lax.fori_loop(0, _CV // _LANES, qbody, 0, unroll=False)
                out_copy(c, p).start()

        out_copy(0, 0).wait()
        out_copy(0, 1).wait()

    return transpose


def _gather_kernel(n_tokens: int, v: int):
    b_per_w = n_tokens // _NW
    n_chunks = b_per_w // _CHUNK
    mesh = plsc.VectorSubcoreMesh(core_axis_name="c", subcore_axis_name="s")

    @functools.partial(
        pl.kernel,
        mesh=mesh,
        out_type=jax.ShapeDtypeStruct((_NW, b_per_w, _RP), jnp.float32),
        scratch_types=[
            pltpu.VMEM((b_per_w,), jnp.int32),
            pltpu.VMEM((b_per_w, _RP), jnp.float32),
            pltpu.SemaphoreType.DMA,
        ],
        compiler_params=pltpu.CompilerParams(
            use_tc_tiling_on_sc=False, needs_layout_passes=False
        ),
    )
    def gather(table_hbm, idx_hbm, out_hbm, idx_v, rows_v, sem):
        wid = lax.axis_index("s") * _NC + lax.axis_index("c")
        base = wid * b_per_w
        pltpu.sync_copy(idx_hbm.at[pl.ds(base, b_per_w)], idx_v)

        def copy(j):
            return pltpu.make_async_copy(
                table_hbm.at[idx_v.at[pl.ds(j * _CHUNK, _CHUNK)]],
                rows_v.at[pl.ds(j * _CHUNK, _CHUNK)],
                sem,
            )

        def body(j, carry):
            copy(j).start()

            @pl.when(j >= _FIRE)
            def _():
                copy(j - _FIRE).wait()

            return carry

        lax.fori_loop(0, n_chunks, body, 0, unroll=False)
        for j in range(max(n_chunks - _FIRE, 0), n_chunks):
            copy(j).wait()
        pltpu.sync_copy(rows_v, out_hbm.at[wid])

    return gather


def _matmul_call(emb2, w, n_rows: int):
    grid = 8
    bm = n_rows // grid
    n_cols = _TPR * _D

    def body(e_ref, w_ref, o_ref):
        o_ref[...] = jnp.dot(
            e_ref[...], w_ref[...], preferred_element_type=jnp.float32
        )

    return pl.pallas_call(
        body,
        grid=(grid,),
        in_specs=[
            pl.BlockSpec((bm, 128), lambda i: (i, 0)),
            pl.BlockSpec((128, n_cols), lambda i: (0, 0)),
        ],
        out_specs=pl.BlockSpec((bm, n_cols), lambda i: (i, 0)),
        out_shape=jax.ShapeDtypeStruct((n_rows, n_cols), jnp.float32),
    )(emb2, w)


def kernel(x, A, B):
    b, l = x.shape
    n_tokens = b * l
    v = A.shape[1]
    idx = x.reshape(n_tokens).astype(jnp.int32)
    bt8 = jnp.zeros((_RP, _D), jnp.float32).at[:_R, :].set(B.T * _SCALING)
    w = jnp.kron(jnp.eye(_TPR, dtype=jnp.float32), bt8)

    table = _transpose_kernel(v)(A).reshape(v, _RP)
    emb = _gather_kernel(n_tokens, v)(table, idx)
    n_rows = n_tokens * _RP // 128
    emb2 = emb.reshape(n_rows, 128)
    out = _matmul_call(emb2, w, n_rows)
    return out.reshape(b, l, _D)


# speedup vs baseline: 1.4408x; 1.0131x over previous
import functools

import jax
import jax.numpy as jnp
from jax import lax
from jax.experimental import pallas as pl
from jax.experimental.pallas import tpu as pltpu
from jax.experimental.pallas import tpu_sc as plsc

_R = 4
_RP = 8
_D = 64
_SCALING = 1.0 / _R

_NC = 2
_NS = 16
_NW = _NC * _NS

_CHUNK = 128
_FIRE = 16
_LANES = 16
_CV = 4000
_TPR = 128 // _RP


def _transpose_kernel(v: int):
    n_chunks = v // _CV
    n_steps = -(-n_chunks // _NW)
    mesh = plsc.VectorSubcoreMesh(core_axis_name="c", subcore_axis_name="s")

    @functools.partial(
        pl.kernel,
        mesh=mesh,
        out_type=jax.ShapeDtypeStruct((v * _RP,), jnp.float32),
        scratch_types=[
            pltpu.VMEM((2, _R, _CV), jnp.float32),
            pltpu.VMEM((2, _CV * _RP), jnp.float32),
            pltpu.SemaphoreType.DMA,
            pltpu.SemaphoreType.DMA,
            pltpu.SemaphoreType.DMA,
            pltpu.SemaphoreType.DMA,
        ],
        compiler_params=pltpu.CompilerParams(
            use_tc_tiling_on_sc=False, needs_layout_passes=False
        ),
    )
    def transpose(a_hbm, table_hbm, buf_v, slab_v, semg0, semg1, semo0, semo1):
        wid = lax.axis_index("s") * _NC + lax.axis_index("c")
        io8 = lax.broadcasted_iota(jnp.int32, (_LANES,), 0) * _RP
        semg = [semg0, semg1]
        semo = [semo0, semo1]

        zvec = jnp.zeros((_LANES,), jnp.float32)

        def zbody(k, carry):
            for p in range(2):
                slab_v[p, pl.ds(k * _LANES, _LANES)] = zvec
            return carry

        lax.fori_loop(0, _CV * _RP // _LANES, zbody, 0, unroll=False)

        def chunk_of(s):
            return wid + s * _NW

        def load_copy(c, p, r):
            return pltpu.make_async_copy(
                a_hbm.at[r, pl.ds(c * _CV, _CV)], buf_v.at[p, r], semg[p]
            )

        def out_copy(c, p):
            return pltpu.make_async_copy(
                slab_v.at[p],
                table_hbm.at[pl.ds(c * _CV * _RP, _CV * _RP)],
                semo[p],
            )

        def fire_loads(s, p):
            c = chunk_of(s)

            @pl.when(c < n_chunks)
            def _():
                for r in range(_R):
                    load_copy(c, p, r).start()

        fire_loads(0, 0)
        for s in range(n_steps):
            p = s % 2
            c = chunk_of(s)
            fire_loads(s + 1, 1 - p)

            @pl.when(c < n_chunks)
            def _():
                for r in range(_R):
                    load_copy(c, p, r).wait()
                if s >= 2:
                    out_copy(c, p).wait()

                def qbody(q, carry):
                    for r in range(_R):
                        plsc.store_scatter(
                            slab_v.at[p],
                            [io8 + (q * _LANES * _RP + r)],
                            buf_v[p, r, pl.ds(q * _LANES, _LANES)],
                        )
                    return carry

                lax.fori_loop(0, _CV // _LANES, qbody, 0, unroll=2)
                out_copy(c, p).start()

        out_copy(0, 0).wait()
        out_copy(0, 1).wait()

    return transpose


def _gather_kernel(n_tokens: int, v: int):
    b_per_w = n_tokens // _NW
    n_chunks = b_per_w // _CHUNK
    mesh = plsc.VectorSubcoreMesh(core_axis_name="c", subcore_axis_name="s")

    @functools.partial(
        pl.kernel,
        mesh=mesh,
        out_type=jax.ShapeDtypeStruct((_NW, b_per_w, _RP), jnp.float32),
        scratch_types=[
            pltpu.VMEM((b_per_w,), jnp.int32),
            pltpu.VMEM((b_per_w, _RP), jnp.float32),
            pltpu.SemaphoreType.DMA,
        ],
        compiler_params=pltpu.CompilerParams(
            use_tc_tiling_on_sc=False, needs_layout_passes=False
        ),
    )
    def gather(table_hbm, idx_hbm, out_hbm, idx_v, rows_v, sem):
        wid = lax.axis_index("s") * _NC + lax.axis_index("c")
        base = wid * b_per_w
        pltpu.sync_copy(idx_hbm.at[pl.ds(base, b_per_w)], idx_v)

        def copy(j):
            return pltpu.make_async_copy(
                table_hbm.at[idx_v.at[pl.ds(j * _CHUNK, _CHUNK)]],
                rows_v.at[pl.ds(j * _CHUNK, _CHUNK)],
                sem,
            )

        def body(j, carry):
            copy(j).start()

            @pl.when(j >= _FIRE)
            def _():
                copy(j - _FIRE).wait()

            return carry

        lax.fori_loop(0, n_chunks, body, 0, unroll=False)
        for j in range(max(n_chunks - _FIRE, 0), n_chunks):
            copy(j).wait()
        pltpu.sync_copy(rows_v, out_hbm.at[wid])

    return gather


def _matmul_call(emb2, w, n_rows: int):
    grid = 8
    bm = n_rows // grid
    n_cols = _TPR * _D

    def body(e_ref, w_ref, o_ref):
        o_ref[...] = jnp.dot(
            e_ref[...], w_ref[...], preferred_element_type=jnp.float32
        )

    return pl.pallas_call(
        body,
        grid=(grid,),
        in_specs=[
            pl.BlockSpec((bm, 128), lambda i: (i, 0)),
            pl.BlockSpec((128, n_cols), lambda i: (0, 0)),
        ],
        out_specs=pl.BlockSpec((bm, n_cols), lambda i: (i, 0)),
        out_shape=jax.ShapeDtypeStruct((n_rows, n_cols), jnp.float32),
    )(emb2, w)


def kernel(x, A, B):
    b, l = x.shape
    n_tokens = b * l
    v = A.shape[1]
    idx = x.reshape(n_tokens).astype(jnp.int32)
    bt8 = jnp.zeros((_RP, _D), jnp.float32).at[:_R, :].set(B.T * _SCALING)
    w = jnp.kron(jnp.eye(_TPR, dtype=jnp.float32), bt8)

    table = _transpose_kernel(v)(A).reshape(v, _RP)
    emb = _gather_kernel(n_tokens, v)(table, idx)
    n_rows = n_tokens * _RP // 128
    emb2 = emb.reshape(n_rows, 128)
    out = _matmul_call(emb2, w, n_rows)
    return out.reshape(b, l, _D)
